# Initial kernel scaffold; baseline (speedup 1.0000x reference)
import jax
import jax.numpy as jnp
from jax.experimental import pallas as pl


def kernel(atom_types, segment_ids, emb_table):
    raise NotImplementedError("write your pallas kernel here")


# speedup vs baseline: 26.6624x; 26.6624x over previous
import functools

import jax
import jax.numpy as jnp
from jax import lax
from jax.experimental import pallas as pl
from jax.experimental.pallas import tpu as pltpu
from jax.experimental.pallas import tpu_sc as plsc

N_OUT = 128
NUM_ATOM_TYPES = 100
N_ATOMS = 320000
N_SEGMENTS = 10000

TYPES_PAD = 128
PADDED = N_SEGMENTS * TYPES_PAD

NC = 2
NS = 16
NW = NC * NS
ATOMS_PER_TILE = N_ATOMS // NW
CHUNK = 128
NCHUNK = -(-ATOMS_PER_TILE // CHUNK)
PAD_TAIL = NCHUNK * CHUNK - ATOMS_PER_TILE
K = PADDED // NS
DUMP = NUM_ATOM_TYPES
LANES = 16


def _sc_histogram(atom_types, segment_ids, zeros):
    mesh = plsc.VectorSubcoreMesh(core_axis_name="c", subcore_axis_name="s")

    @functools.partial(
        pl.kernel,
        out_type=jax.ShapeDtypeStruct((NC, PADDED), jnp.float32),
        mesh=mesh,
        scratch_types=[
            pltpu.VMEM_SHARED((PADDED,), jnp.float32),
            pltpu.VMEM((ATOMS_PER_TILE,), jnp.int32),
            pltpu.VMEM((ATOMS_PER_TILE,), jnp.int32),
            pltpu.VMEM((NCHUNK, CHUNK), jnp.int32),
            pltpu.VMEM((CHUNK,), jnp.float32),
        ],
    )
    def hist(typ_hbm, seg_hbm, zero_hbm, counts_hbm, shared, seg_v, typ_v,
             idx_v, ones_v):
        cid = lax.axis_index("c")
        sid = lax.axis_index("s")
        wid = cid * NS + sid
        base_atom = wid * ATOMS_PER_TILE

        pltpu.sync_copy(seg_hbm.at[pl.ds(base_atom, ATOMS_PER_TILE)], seg_v)
        pltpu.sync_copy(typ_hbm.at[pl.ds(base_atom, ATOMS_PER_TILE)], typ_v)
        pltpu.sync_copy(zero_hbm.at[pl.ds(sid * K, K)],
                        shared.at[pl.ds(sid * K, K)])

        one = jnp.full((LANES,), 1.0, jnp.float32)
        for c in range(CHUNK // LANES):
            ones_v[pl.ds(c * LANES, LANES)] = one

        @pl.loop(0, NCHUNK - 1)
        def _(j):
            for c in range(CHUNK // LANES):
                off = j * CHUNK + c * LANES
                s16 = seg_v[pl.ds(off, LANES)]
                t16 = typ_v[pl.ds(off, LANES)]
                idx_v[j, pl.ds(c * LANES, LANES)] = s16 * TYPES_PAD + t16

        last = NCHUNK - 1
        n_valid = (ATOMS_PER_TILE - last * CHUNK) // LANES
        dump = jnp.full((LANES,), DUMP, jnp.int32)
        for c in range(CHUNK // LANES):
            if c < n_valid:
                off = last * CHUNK + c * LANES
                s16 = seg_v[pl.ds(off, LANES)]
                t16 = typ_v[pl.ds(off, LANES)]
                idx_v[last, pl.ds(c * LANES, LANES)] = s16 * TYPES_PAD + t16
            else:
                idx_v[last, pl.ds(c * LANES, LANES)] = dump

        plsc.subcore_barrier()

        @pl.loop(0, NCHUNK)
        def _(j):
            pltpu.sync_copy(ones_v, shared.at[idx_v.at[j]], add=True)

        plsc.subcore_barrier()

        pltpu.sync_copy(shared.at[pl.ds(sid * K, K)],
                        counts_hbm.at[cid, pl.ds(sid * K, K)])

    return hist(atom_types, segment_ids, zeros)


def _tc_matmul(counts3, table_pad):
    rows = 1000

    def body(c_ref, t_ref, o_ref):
        a = c_ref[0] + c_ref[1]
        o_ref[...] = jnp.dot(a, t_ref[...], preferred_element_type=jnp.float32)

    return pl.pallas_call(
        body,
        grid=(N_SEGMENTS // rows,),
        in_specs=[
            pl.BlockSpec((NC, rows, TYPES_PAD), lambda i: (0, i, 0)),
            pl.BlockSpec((TYPES_PAD, N_OUT), lambda i: (0, 0)),
        ],
        out_specs=pl.BlockSpec((rows, N_OUT), lambda i: (i, 0)),
        out_shape=jax.ShapeDtypeStruct((N_SEGMENTS, N_OUT), jnp.float32),
    )(counts3, table_pad)


def kernel(atom_types, segment_ids, emb_table):
    zeros = jnp.zeros((PADDED,), jnp.float32)
    table_pad = jnp.zeros((TYPES_PAD, N_OUT), jnp.float32)
    table_pad = table_pad.at[:NUM_ATOM_TYPES].set(emb_table)
    counts = _sc_histogram(atom_types.astype(jnp.int32),
                           segment_ids.astype(jnp.int32), zeros)
    counts3 = counts.reshape(NC, N_SEGMENTS, TYPES_PAD)
    return _tc_matmul(counts3, table_pad)


# speedup vs baseline: 32.4892x; 1.2185x over previous
import functools

import jax
import jax.numpy as jnp
from jax import lax
from jax.experimental import pallas as pl
from jax.experimental.pallas import tpu as pltpu
from jax.experimental.pallas import tpu_sc as plsc

N_OUT = 128
NUM_ATOM_TYPES = 100
N_ATOMS = 320000
N_SEGMENTS = 10000

TYPES_PAD = 128
PADDED = N_SEGMENTS * TYPES_PAD

NC = 2
NS = 16
NW = NC * NS
ATOMS_PER_TILE = N_ATOMS // NW
CHUNK = 128
NCHUNK = -(-ATOMS_PER_TILE // CHUNK)
PAD_TAIL = NCHUNK * CHUNK - ATOMS_PER_TILE
ROWS_PER_TILE = N_SEGMENTS // NS
DUMP = NUM_ATOM_TYPES
LANES = 16


def _sc_histogram(atom_types, segment_ids, zeros):
    mesh = plsc.VectorSubcoreMesh(core_axis_name="c", subcore_axis_name="s")

    @functools.partial(
        pl.kernel,
        out_type=[jax.ShapeDtypeStruct((PADDED,), jnp.float32),
                  jax.ShapeDtypeStruct((PADDED,), jnp.float32)],
        mesh=mesh,
        scratch_types=[
            pltpu.VMEM_SHARED((PADDED,), jnp.float32),
            pltpu.VMEM((ATOMS_PER_TILE,), jnp.int32),
            pltpu.VMEM((ATOMS_PER_TILE,), jnp.int32),
            pltpu.VMEM((NCHUNK, CHUNK), jnp.int32),
            pltpu.VMEM((CHUNK,), jnp.float32),
        ],
    )
    def hist(typ_hbm, seg_hbm, zero_hbm, counts0_hbm, counts1_hbm, shared,
             seg_v, typ_v, idx_v, ones_v):
        cid = lax.axis_index("c")
        sid = lax.axis_index("s")
        wid = cid * NS + sid
        base_atom = wid * ATOMS_PER_TILE

        pltpu.sync_copy(seg_hbm.at[pl.ds(base_atom, ATOMS_PER_TILE)], seg_v)
        pltpu.sync_copy(typ_hbm.at[pl.ds(base_atom, ATOMS_PER_TILE)], typ_v)
        word0 = sid * (PADDED // NS)
        pltpu.sync_copy(zero_hbm.at[pl.ds(word0, PADDED // NS)],
                        shared.at[pl.ds(word0, PADDED // NS)])

        one = jnp.full((LANES,), 1.0, jnp.float32)
        for c in range(CHUNK // LANES):
            ones_v[pl.ds(c * LANES, LANES)] = one

        @pl.loop(0, NCHUNK - 1)
        def _(j):
            for c in range(CHUNK // LANES):
                off = j * CHUNK + c * LANES
                s16 = seg_v[pl.ds(off, LANES)]
                t16 = typ_v[pl.ds(off, LANES)]
                idx_v[j, pl.ds(c * LANES, LANES)] = s16 * TYPES_PAD + t16

        last = NCHUNK - 1
        n_valid = (ATOMS_PER_TILE - last * CHUNK) // LANES
        dump = jnp.full((LANES,), DUMP, jnp.int32)
        for c in range(CHUNK // LANES):
            if c < n_valid:
                off = last * CHUNK + c * LANES
                s16 = seg_v[pl.ds(off, LANES)]
                t16 = typ_v[pl.ds(off, LANES)]
                idx_v[last, pl.ds(c * LANES, LANES)] = s16 * TYPES_PAD + t16
            else:
                idx_v[last, pl.ds(c * LANES, LANES)] = dump

        plsc.subcore_barrier()

        @pl.loop(0, NCHUNK)
        def _(j):
            pltpu.sync_copy(ones_v, shared.at[idx_v.at[j]], add=True)

        plsc.subcore_barrier()

        @pl.when(cid == 0)
        def _():
            pltpu.sync_copy(shared.at[pl.ds(word0, PADDED // NS)],
                            counts0_hbm.at[pl.ds(word0, PADDED // NS)])

        @pl.when(cid == 1)
        def _():
            pltpu.sync_copy(shared.at[pl.ds(word0, PADDED // NS)],
                            counts1_hbm.at[pl.ds(word0, PADDED // NS)])

    return hist(atom_types, segment_ids, zeros)


def _tc_matmul(c0, c1, table_pad):
    rows = 1000

    def body(c0_ref, c1_ref, t_ref, o_ref):
        a = c0_ref[...] + c1_ref[...]
        o_ref[...] = jnp.dot(a, t_ref[...], preferred_element_type=jnp.float32)

    return pl.pallas_call(
        body,
        grid=(N_SEGMENTS // rows,),
        in_specs=[
            pl.BlockSpec((rows, TYPES_PAD), lambda i: (i, 0)),
            pl.BlockSpec((rows, TYPES_PAD), lambda i: (i, 0)),
            pl.BlockSpec((TYPES_PAD, N_OUT), lambda i: (0, 0)),
        ],
        out_specs=pl.BlockSpec((rows, N_OUT), lambda i: (i, 0)),
        out_shape=jax.ShapeDtypeStruct((N_SEGMENTS, N_OUT), jnp.float32),
    )(c0, c1, table_pad)


def kernel(atom_types, segment_ids, emb_table):
    zeros = jnp.zeros((PADDED,), jnp.float32)
    table_pad = jnp.zeros((TYPES_PAD, N_OUT), jnp.float32)
    table_pad = table_pad.at[:NUM_ATOM_TYPES].set(emb_table)
    counts0, counts1 = _sc_histogram(atom_types.astype(jnp.int32),
                                     segment_ids.astype(jnp.int32), zeros)
    c0 = counts0.reshape(N_SEGMENTS, TYPES_PAD)
    c1 = counts1.reshape(N_SEGMENTS, TYPES_PAD)
    return _tc_matmul(c0, c1, table_pad)


# speedup vs baseline: 35.2295x; 1.0843x over previous
import functools

import jax
import jax.numpy as jnp
from jax import lax
from jax.experimental import pallas as pl
from jax.experimental.pallas import tpu as pltpu
from jax.experimental.pallas import tpu_sc as plsc

N_OUT = 128
NUM_ATOM_TYPES = 100
N_ATOMS = 320000
N_SEGMENTS = 10000

TYPES_PAD = 128
PADDED = N_SEGMENTS * TYPES_PAD

NC = 2
NS = 16
NW = NC * NS
ATOMS_PER_TILE = N_ATOMS // NW
CHUNK = 128
SCAT_W = 8
NCHUNK = 80
PAD_TAIL = NCHUNK * CHUNK - ATOMS_PER_TILE
ROWS_PER_TILE = N_SEGMENTS // NS
DUMP = NUM_ATOM_TYPES
LANES = 16


def _sc_histogram(atom_types, segment_ids, zeros):
    mesh = plsc.VectorSubcoreMesh(core_axis_name="c", subcore_axis_name="s")

    @functools.partial(
        pl.kernel,
        out_type=[jax.ShapeDtypeStruct((PADDED,), jnp.float32),
                  jax.ShapeDtypeStruct((PADDED,), jnp.float32)],
        mesh=mesh,
        scratch_types=[
            pltpu.VMEM_SHARED((PADDED,), jnp.float32),
            pltpu.VMEM((ATOMS_PER_TILE,), jnp.int32),
            pltpu.VMEM((ATOMS_PER_TILE,), jnp.int32),
            pltpu.VMEM((NCHUNK, CHUNK), jnp.int32),
            pltpu.VMEM((CHUNK,), jnp.float32),
            pltpu.SemaphoreType.DMA,
            pltpu.SemaphoreType.DMA,
            pltpu.SemaphoreType.DMA,
        ],
    )
    def hist(typ_hbm, seg_hbm, zero_hbm, counts0_hbm, counts1_hbm, shared,
             seg_v, typ_v, idx_v, ones_v, sem_z, sem_s, sem_t):
        cid = lax.axis_index("c")
        sid = lax.axis_index("s")
        wid = cid * NS + sid
        base_atom = wid * ATOMS_PER_TILE

        word0 = sid * (PADDED // NS)
        zero_cp = pltpu.async_copy(zero_hbm.at[pl.ds(word0, PADDED // NS)],
                                   shared.at[pl.ds(word0, PADDED // NS)],
                                   sem_z)
        seg_cp = pltpu.async_copy(seg_hbm.at[pl.ds(base_atom, ATOMS_PER_TILE)],
                                  seg_v, sem_s)
        typ_cp = pltpu.async_copy(typ_hbm.at[pl.ds(base_atom, ATOMS_PER_TILE)],
                                  typ_v, sem_t)

        one = jnp.full((LANES,), 1.0, jnp.float32)
        for c in range(CHUNK // LANES):
            ones_v[pl.ds(c * LANES, LANES)] = one

        seg_cp.wait()
        typ_cp.wait()

        nfull = ATOMS_PER_TILE // CHUNK

        @pl.loop(0, nfull)
        def _(j):
            for c in range(CHUNK // LANES):
                off = j * CHUNK + c * LANES
                s16 = seg_v[pl.ds(off, LANES)]
                t16 = typ_v[pl.ds(off, LANES)]
                idx_v[j, pl.ds(c * LANES, LANES)] = s16 * TYPES_PAD + t16

        n_valid = (ATOMS_PER_TILE - nfull * CHUNK) // LANES
        dump = jnp.full((LANES,), DUMP, jnp.int32)
        for g in range(n_valid):
            off = nfull * CHUNK + g * LANES
            s16 = seg_v[pl.ds(off, LANES)]
            t16 = typ_v[pl.ds(off, LANES)]
            idx_v[nfull, pl.ds(g * LANES, LANES)] = s16 * TYPES_PAD + t16
        for j in range(nfull, NCHUNK):
            for c in range(CHUNK // LANES):
                if j == nfull and c < n_valid:
                    continue
                idx_v[j, pl.ds(c * LANES, LANES)] = dump

        zero_cp.wait()
        plsc.subcore_barrier()

        @pl.loop(0, NCHUNK // SCAT_W)
        def _(jw):
            for b in range(SCAT_W):
                pltpu.make_async_copy(ones_v,
                                      shared.at[idx_v.at[jw * SCAT_W + b]],
                                      sem_z).start(add=True)
            for b in range(SCAT_W):
                pltpu.make_async_copy(ones_v,
                                      shared.at[idx_v.at[jw * SCAT_W + b]],
                                      sem_z).wait()

        plsc.subcore_barrier()

        @pl.when(cid == 0)
        def _():
            pltpu.sync_copy(shared.at[pl.ds(word0, PADDED // NS)],
                            counts0_hbm.at[pl.ds(word0, PADDED // NS)])

        @pl.when(cid == 1)
        def _():
            pltpu.sync_copy(shared.at[pl.ds(word0, PADDED // NS)],
                            counts1_hbm.at[pl.ds(word0, PADDED // NS)])

    return hist(atom_types, segment_ids, zeros)


def _tc_matmul(c0, c1, table_pad):
    rows = 1000

    def body(c0_ref, c1_ref, t_ref, o_ref):
        a = c0_ref[...] + c1_ref[...]
        o_ref[...] = jnp.dot(a, t_ref[...], preferred_element_type=jnp.float32)

    return pl.pallas_call(
        body,
        grid=(N_SEGMENTS // rows,),
        in_specs=[
            pl.BlockSpec((rows, TYPES_PAD), lambda i: (i, 0)),
            pl.BlockSpec((rows, TYPES_PAD), lambda i: (i, 0)),
            pl.BlockSpec((TYPES_PAD, N_OUT), lambda i: (0, 0)),
        ],
        out_specs=pl.BlockSpec((rows, N_OUT), lambda i: (i, 0)),
        out_shape=jax.ShapeDtypeStruct((N_SEGMENTS, N_OUT), jnp.float32),
    )(c0, c1, table_pad)


def kernel(atom_types, segment_ids, emb_table):
    zeros = jnp.zeros((PADDED,), jnp.float32)
    table_pad = jnp.zeros((TYPES_PAD, N_OUT), jnp.float32)
    table_pad = table_pad.at[:NUM_ATOM_TYPES].set(emb_table)
    counts0, counts1 = _sc_histogram(atom_types.astype(jnp.int32),
                                     segment_ids.astype(jnp.int32), zeros)
    c0 = counts0.reshape(N_SEGMENTS, TYPES_PAD)
    c1 = counts1.reshape(N_SEGMENTS, TYPES_PAD)
    return _tc_matmul(c0, c1, table_pad)


# speedup vs baseline: 39.8790x; 1.1320x over previous
import functools

import jax
import jax.numpy as jnp
from jax import lax
from jax.experimental import pallas as pl
from jax.experimental.pallas import tpu as pltpu
from jax.experimental.pallas import tpu_sc as plsc

N_OUT = 128
NUM_ATOM_TYPES = 100
N_ATOMS = 320000
N_SEGMENTS = 10000

TYPES_PAD = 128
PADDED = N_SEGMENTS * TYPES_PAD

NC = 2
NS = 16
NW = NC * NS
ATOMS_PER_TILE = N_ATOMS // NW
CHUNK = 128
SCAT_W = 8
NCHUNK = 80
PAD_TAIL = NCHUNK * CHUNK - ATOMS_PER_TILE
ROWS_PER_TILE = N_SEGMENTS // NS
DUMP = NUM_ATOM_TYPES
LANES = 16


ZBUF = 4000


def _sc_histogram(atom_types, segment_ids):
    mesh = plsc.VectorSubcoreMesh(core_axis_name="c", subcore_axis_name="s")

    @functools.partial(
        pl.kernel,
        out_type=[jax.ShapeDtypeStruct((PADDED,), jnp.float32),
                  jax.ShapeDtypeStruct((PADDED,), jnp.float32)],
        mesh=mesh,
        scratch_types=[
            pltpu.VMEM_SHARED((PADDED,), jnp.float32),
            pltpu.VMEM((ATOMS_PER_TILE,), jnp.int32),
            pltpu.VMEM((ATOMS_PER_TILE,), jnp.int32),
            pltpu.VMEM((NCHUNK, CHUNK), jnp.int32),
            pltpu.VMEM((CHUNK,), jnp.float32),
            pltpu.VMEM((ZBUF,), jnp.float32),
            pltpu.SemaphoreType.DMA,
            pltpu.SemaphoreType.DMA,
            pltpu.SemaphoreType.DMA,
        ],
    )
    def hist(typ_hbm, seg_hbm, counts0_hbm, counts1_hbm, shared,
             seg_v, typ_v, idx_v, ones_v, zbuf, sem_z, sem_s, sem_t):
        cid = lax.axis_index("c")
        sid = lax.axis_index("s")
        wid = cid * NS + sid
        base_atom = wid * ATOMS_PER_TILE
        word0 = sid * (PADDED // NS)

        seg_cp = pltpu.async_copy(seg_hbm.at[pl.ds(base_atom, ATOMS_PER_TILE)],
                                  seg_v, sem_s)
        typ_cp = pltpu.async_copy(typ_hbm.at[pl.ds(base_atom, ATOMS_PER_TILE)],
                                  typ_v, sem_t)

        zero16 = jnp.zeros((LANES,), jnp.float32)

        @pl.loop(0, ZBUF // LANES)
        def _(i):
            zbuf[pl.ds(i * LANES, LANES)] = zero16

        nz = (PADDED // NS) // ZBUF

        @pl.loop(0, nz)
        def _(k):
            pltpu.make_async_copy(
                zbuf, shared.at[pl.ds(word0 + k * ZBUF, ZBUF)], sem_z).start()

        one = jnp.full((LANES,), 1.0, jnp.float32)
        for c in range(CHUNK // LANES):
            ones_v[pl.ds(c * LANES, LANES)] = one

        seg_cp.wait()
        typ_cp.wait()

        nfull = ATOMS_PER_TILE // CHUNK

        @pl.loop(0, nfull)
        def _(j):
            for c in range(CHUNK // LANES):
                off = j * CHUNK + c * LANES
                s16 = seg_v[pl.ds(off, LANES)]
                t16 = typ_v[pl.ds(off, LANES)]
                idx_v[j, pl.ds(c * LANES, LANES)] = s16 * TYPES_PAD + t16

        n_valid = (ATOMS_PER_TILE - nfull * CHUNK) // LANES
        dump = jnp.full((LANES,), DUMP, jnp.int32)
        for g in range(n_valid):
            off = nfull * CHUNK + g * LANES
            s16 = seg_v[pl.ds(off, LANES)]
            t16 = typ_v[pl.ds(off, LANES)]
            idx_v[nfull, pl.ds(g * LANES, LANES)] = s16 * TYPES_PAD + t16
        for j in range(nfull, NCHUNK):
            for c in range(CHUNK // LANES):
                if j == nfull and c < n_valid:
                    continue
                idx_v[j, pl.ds(c * LANES, LANES)] = dump

        @pl.loop(0, nz)
        def _(k):
            pltpu.make_async_copy(
                zbuf, shared.at[pl.ds(word0 + k * ZBUF, ZBUF)], sem_z).wait()

        plsc.subcore_barrier()

        @pl.loop(0, NCHUNK // SCAT_W)
        def _(jw):
            for b in range(SCAT_W):
                pltpu.make_async_copy(ones_v,
                                      shared.at[idx_v.at[jw * SCAT_W + b]],
                                      sem_z).start(add=True)
            for b in range(SCAT_W):
                pltpu.make_async_copy(ones_v,
                                      shared.at[idx_v.at[jw * SCAT_W + b]],
                                      sem_z).wait()

        plsc.subcore_barrier()

        @pl.when(cid == 0)
        def _():
            pltpu.sync_copy(shared.at[pl.ds(word0, PADDED // NS)],
                            counts0_hbm.at[pl.ds(word0, PADDED // NS)])

        @pl.when(cid == 1)
        def _():
            pltpu.sync_copy(shared.at[pl.ds(word0, PADDED // NS)],
                            counts1_hbm.at[pl.ds(word0, PADDED // NS)])

    return hist(atom_types, segment_ids)


def _tc_matmul(c0, c1, table_pad):
    rows = 2000

    def body(c0_ref, c1_ref, t_ref, o_ref):
        a = c0_ref[...] + c1_ref[...]
        o_ref[...] = jnp.dot(a, t_ref[...], preferred_element_type=jnp.float32)

    return pl.pallas_call(
        body,
        grid=(N_SEGMENTS // rows,),
        in_specs=[
            pl.BlockSpec((rows, TYPES_PAD), lambda i: (i, 0)),
            pl.BlockSpec((rows, TYPES_PAD), lambda i: (i, 0)),
            pl.BlockSpec((TYPES_PAD, N_OUT), lambda i: (0, 0)),
        ],
        out_specs=pl.BlockSpec((rows, N_OUT), lambda i: (i, 0)),
        out_shape=jax.ShapeDtypeStruct((N_SEGMENTS, N_OUT), jnp.float32),
    )(c0, c1, table_pad)


def kernel(atom_types, segment_ids, emb_table):
    table_pad = jnp.zeros((TYPES_PAD, N_OUT), jnp.float32)
    table_pad = table_pad.at[:NUM_ATOM_TYPES].set(emb_table)
    counts0, counts1 = _sc_histogram(atom_types.astype(jnp.int32),
                                     segment_ids.astype(jnp.int32))
    c0 = counts0.reshape(N_SEGMENTS, TYPES_PAD)
    c1 = counts1.reshape(N_SEGMENTS, TYPES_PAD)
    return _tc_matmul(c0, c1, table_pad)


# speedup vs baseline: 40.0700x; 1.0048x over previous
import functools

import jax
import jax.numpy as jnp
from jax import lax
from jax.experimental import pallas as pl
from jax.experimental.pallas import tpu as pltpu
from jax.experimental.pallas import tpu_sc as plsc

N_OUT = 128
NUM_ATOM_TYPES = 100
N_ATOMS = 320000
N_SEGMENTS = 10000

TYPES_PAD = 128
PADDED = N_SEGMENTS * TYPES_PAD

NC = 2
NS = 16
NW = NC * NS
ATOMS_PER_TILE = N_ATOMS // NW
CHUNK = 128
SCAT_W = 8
NCHUNK = 80
PAD_TAIL = NCHUNK * CHUNK - ATOMS_PER_TILE
ROWS_PER_TILE = N_SEGMENTS // NS
DUMP = NUM_ATOM_TYPES
LANES = 16


ZBUF = 4000


def _sc_histogram(atom_types, segment_ids):
    mesh = plsc.VectorSubcoreMesh(core_axis_name="c", subcore_axis_name="s")

    @functools.partial(
        pl.kernel,
        out_type=[jax.ShapeDtypeStruct((PADDED,), jnp.float32),
                  jax.ShapeDtypeStruct((PADDED,), jnp.float32)],
        mesh=mesh,
        scratch_types=[
            pltpu.VMEM_SHARED((PADDED,), jnp.float32),
            pltpu.VMEM((ATOMS_PER_TILE,), jnp.int32),
            pltpu.VMEM((ATOMS_PER_TILE,), jnp.int32),
            pltpu.VMEM((NCHUNK, CHUNK), jnp.int32),
            pltpu.VMEM((CHUNK,), jnp.float32),
            pltpu.VMEM((ZBUF,), jnp.float32),
            pltpu.SemaphoreType.DMA,
            pltpu.SemaphoreType.DMA,
            pltpu.SemaphoreType.DMA,
        ],
    )
    def hist(typ_hbm, seg_hbm, counts0_hbm, counts1_hbm, shared,
             seg_v, typ_v, idx_v, ones_v, zbuf, sem_z, sem_s, sem_t):
        cid = lax.axis_index("c")
        sid = lax.axis_index("s")
        wid = cid * NS + sid
        base_atom = wid * ATOMS_PER_TILE
        word0 = sid * (PADDED // NS)

        seg_cp = pltpu.async_copy(seg_hbm.at[pl.ds(base_atom, ATOMS_PER_TILE)],
                                  seg_v, sem_s)
        typ_cp = pltpu.async_copy(typ_hbm.at[pl.ds(base_atom, ATOMS_PER_TILE)],
                                  typ_v, sem_t)

        zero16 = jnp.zeros((LANES,), jnp.float32)

        @pl.loop(0, ZBUF // LANES)
        def _(i):
            zbuf[pl.ds(i * LANES, LANES)] = zero16

        nz = (PADDED // NS) // ZBUF

        @pl.loop(0, nz)
        def _(k):
            pltpu.make_async_copy(
                zbuf, shared.at[pl.ds(word0 + k * ZBUF, ZBUF)], sem_z).start()

        one = jnp.full((LANES,), 1.0, jnp.float32)
        for c in range(CHUNK // LANES):
            ones_v[pl.ds(c * LANES, LANES)] = one

        seg_cp.wait()
        typ_cp.wait()

        nfull = ATOMS_PER_TILE // CHUNK

        @pl.loop(0, nfull)
        def _(j):
            for c in range(CHUNK // LANES):
                off = j * CHUNK + c * LANES
                s16 = seg_v[pl.ds(off, LANES)]
                t16 = typ_v[pl.ds(off, LANES)]
                idx_v[j, pl.ds(c * LANES, LANES)] = s16 * TYPES_PAD + t16

        n_valid = (ATOMS_PER_TILE - nfull * CHUNK) // LANES
        dump = jnp.full((LANES,), DUMP, jnp.int32)
        for g in range(n_valid):
            off = nfull * CHUNK + g * LANES
            s16 = seg_v[pl.ds(off, LANES)]
            t16 = typ_v[pl.ds(off, LANES)]
            idx_v[nfull, pl.ds(g * LANES, LANES)] = s16 * TYPES_PAD + t16
        for j in range(nfull, NCHUNK):
            for c in range(CHUNK // LANES):
                if j == nfull and c < n_valid:
                    continue
                idx_v[j, pl.ds(c * LANES, LANES)] = dump

        @pl.loop(0, nz)
        def _(k):
            pltpu.make_async_copy(
                zbuf, shared.at[pl.ds(word0 + k * ZBUF, ZBUF)], sem_z).wait()

        plsc.subcore_barrier()

        @pl.loop(0, NCHUNK)
        def _(j):
            pltpu.make_async_copy(ones_v, shared.at[idx_v.at[j]],
                                  sem_z).start(add=True)

        @pl.loop(0, NCHUNK)
        def _(j):
            pltpu.make_async_copy(ones_v, shared.at[idx_v.at[j]],
                                  sem_z).wait()

        plsc.subcore_barrier()

        @pl.when(cid == 0)
        def _():
            pltpu.sync_copy(shared.at[pl.ds(word0, PADDED // NS)],
                            counts0_hbm.at[pl.ds(word0, PADDED // NS)])

        @pl.when(cid == 1)
        def _():
            pltpu.sync_copy(shared.at[pl.ds(word0, PADDED // NS)],
                            counts1_hbm.at[pl.ds(word0, PADDED // NS)])

    return hist(atom_types, segment_ids)


def _tc_matmul(c0, c1, table_pad):
    rows = 2000

    def body(c0_ref, c1_ref, t_ref, o_ref):
        a = c0_ref[...] + c1_ref[...]
        o_ref[...] = jnp.dot(a, t_ref[...], preferred_element_type=jnp.float32)

    return pl.pallas_call(
        body,
        grid=(N_SEGMENTS // rows,),
        in_specs=[
            pl.BlockSpec((rows, TYPES_PAD), lambda i: (i, 0)),
            pl.BlockSpec((rows, TYPES_PAD), lambda i: (i, 0)),
            pl.BlockSpec((TYPES_PAD, N_OUT), lambda i: (0, 0)),
        ],
        out_specs=pl.BlockSpec((rows, N_OUT), lambda i: (i, 0)),
        out_shape=jax.ShapeDtypeStruct((N_SEGMENTS, N_OUT), jnp.float32),
    )(c0, c1, table_pad)


def kernel(atom_types, segment_ids, emb_table):
    table_pad = jnp.zeros((TYPES_PAD, N_OUT), jnp.float32)
    table_pad = table_pad.at[:NUM_ATOM_TYPES].set(emb_table)
    counts0, counts1 = _sc_histogram(atom_types.astype(jnp.int32),
                                     segment_ids.astype(jnp.int32))
    c0 = counts0.reshape(N_SEGMENTS, TYPES_PAD)
    c1 = counts1.reshape(N_SEGMENTS, TYPES_PAD)
    return _tc_matmul(c0, c1, table_pad)
